# Initial kernel scaffold; baseline (speedup 1.0000x reference)
import jax
import jax.numpy as jnp
from jax.experimental import pallas as pl


def kernel(h, edge_index0, edge_index1, edge_weight0, edge_weight1, node_type, W0, a0, Wg0, bg0, Wb0, bb0, bias0, W1, a1, Wg1, bg1, Wb1, bb1, bias1, attW1, attb1, attW2):
    raise NotImplementedError("write your pallas kernel here")


# speedup vs baseline: 3.3063x; 3.3063x over previous
import functools

import jax
import jax.numpy as jnp
from jax import lax
from jax.experimental import pallas as pl
from jax.experimental.pallas import tpu as pltpu
from jax.experimental.pallas import tpu_sc as plsc

N = 10000
D = 128
E = 320000

NTILE = 16
CHUNK = 128
BCH = 8
NBLK = 20
NCH = NBLK * BCH
EPT = NCH * CHUNK
EPAD = EPT * NTILE
NPAD = 10240
ROWS_PT = NPAD // NTILE

BR = 2000
NB = N // BR


def _mm_body(h_ref, w_ref, o_ref):
    o_ref[0] = jnp.dot(h_ref[...], w_ref[0], preferred_element_type=jnp.float32)
    o_ref[1] = jnp.dot(h_ref[...], w_ref[1], preferred_element_type=jnp.float32)


def _seq_fts(h, Ws):
    return pl.pallas_call(
        _mm_body,
        grid=(NB,),
        in_specs=[
            pl.BlockSpec((BR, D), lambda i: (i, 0)),
            pl.BlockSpec((2, D, D), lambda i: (0, 0, 0)),
        ],
        out_specs=pl.BlockSpec((2, BR, D), lambda i: (0, i, 0)),
        out_shape=jax.ShapeDtypeStruct((2, N, D), jnp.float32),
    )(h, Ws)


def _sc_body(seq_hbm, src_hbm, dst_hbm, ew_hbm, z_hbm, out_hbm,
             src_v, dst_v, ew_v, rows_v, acc, sem):
    c = lax.axis_index("c")
    s = lax.axis_index("s")

    pltpu.sync_copy(z_hbm.at[pl.ds(s * ROWS_PT, ROWS_PT)],
                    acc.at[pl.ds(s * ROWS_PT, ROWS_PT)])
    plsc.subcore_barrier()

    def block(b, _):
        pltpu.sync_copy(src_hbm.at[c, s, b], src_v)
        pltpu.sync_copy(dst_hbm.at[c, s, b], dst_v)
        pltpu.sync_copy(ew_hbm.at[c, s, b], ew_v)

        def chunk(j, _):
            pltpu.async_copy(seq_hbm.at[src_v.at[j]], rows_v, sem).wait()

            def scale(g, _):
                ew16 = ew_v[j, pl.ds(g * 16, 16)]
                for l in range(16):
                    w = ew16[l]
                    r = g * 16 + l
                    for q in range(8):
                        sl = pl.ds(q * 16, 16)
                        rows_v[r, sl] = rows_v[r, sl] * w
                return 0

            lax.fori_loop(0, CHUNK // 16, scale, 0)
            pltpu.sync_copy(rows_v, acc.at[dst_v.at[j]], add=True)
            return 0

        lax.fori_loop(0, BCH, chunk, 0)
        return 0

    lax.fori_loop(0, NBLK, block, 0)
    plsc.subcore_barrier()
    pltpu.sync_copy(acc.at[pl.ds(s * ROWS_PT, ROWS_PT)],
                    out_hbm.at[c, pl.ds(s * ROWS_PT, ROWS_PT)])


def _sc_agg(seq2, src_s, dst_s, ew_s, zeros):
    mesh = plsc.VectorSubcoreMesh(core_axis_name="c", subcore_axis_name="s")
    k = functools.partial(
        pl.kernel,
        out_type=jax.ShapeDtypeStruct((2, NPAD, D), jnp.float32),
        mesh=mesh,
        scratch_types=[
            pltpu.VMEM((BCH, CHUNK), jnp.int32),
            pltpu.VMEM((BCH, CHUNK), jnp.int32),
            pltpu.VMEM((BCH, CHUNK), jnp.float32),
            pltpu.VMEM((CHUNK, D), jnp.float32),
            pltpu.VMEM_SHARED((NPAD, D), jnp.float32),
            pltpu.SemaphoreType.DMA,
        ],
    )(_sc_body)
    return k(seq2, src_s, dst_s, ew_s, zeros)


def _film_body(h_ref, seq_ref, agg_ref, nt_ref, Wg_ref, Wb_ref, bv_ref,
               a_ref, aW1_ref, ab1_ref, aW2_ref, o_ref):
    nt = nt_ref[...]
    is0 = nt < 0.5
    zs = []
    for i in range(2):
        gam = jnp.where(is0, Wg_ref[i, 0][None, :], Wg_ref[i, 1][None, :])
        bet = jnp.where(is0, Wb_ref[i, 0][None, :], Wb_ref[i, 1][None, :])
        gam = gam + bv_ref[3 * i][None, :]
        bet = bet + bv_ref[3 * i + 1][None, :]
        z = gam * agg_ref[i] + bet + bv_ref[3 * i + 2][None, :] + seq_ref[i]
        z = jnp.where(z >= 0, z, a_ref[i] * z)
        zs.append(z)
    z0, z1 = zs
    aW1 = aW1_ref[...]
    ab1 = ab1_ref[...]
    aW2 = aW2_ref[...]
    w0 = jnp.dot(jnp.tanh(jnp.dot(z0, aW1, preferred_element_type=jnp.float32)
                          + ab1), aW2, preferred_element_type=jnp.float32)
    w1 = jnp.dot(jnp.tanh(jnp.dot(z1, aW1, preferred_element_type=jnp.float32)
                          + ab1), aW2, preferred_element_type=jnp.float32)
    m = jnp.maximum(w0, w1)
    e0 = jnp.exp(w0 - m)
    e1 = jnp.exp(w1 - m)
    inv = 1.0 / (e0 + e1)
    o_ref[...] = (e0 * inv) * z0 + (e1 * inv) * z1 + h_ref[...]


def _film_att(h, seq, agg, nt_f, Wg_s, Wb_s, bv, a_s, attW1, attb1, attW2):
    return pl.pallas_call(
        _film_body,
        grid=(NB,),
        in_specs=[
            pl.BlockSpec((BR, D), lambda i: (i, 0)),
            pl.BlockSpec((2, BR, D), lambda i: (0, i, 0)),
            pl.BlockSpec((2, BR, D), lambda i: (0, i, 0)),
            pl.BlockSpec((BR, 1), lambda i: (i, 0)),
            pl.BlockSpec((2, 2, D), lambda i: (0, 0, 0)),
            pl.BlockSpec((2, 2, D), lambda i: (0, 0, 0)),
            pl.BlockSpec((8, D), lambda i: (0, 0)),
            pl.BlockSpec(memory_space=pltpu.SMEM),
            pl.BlockSpec((D, D), lambda i: (0, 0)),
            pl.BlockSpec((1, D), lambda i: (0, 0)),
            pl.BlockSpec((D, 1), lambda i: (0, 0)),
        ],
        out_specs=pl.BlockSpec((BR, D), lambda i: (i, 0)),
        out_shape=jax.ShapeDtypeStruct((N, D), jnp.float32),
    )(h, seq, agg, nt_f, Wg_s, Wb_s, bv, a_s, attW1, attb1, attW2)


def _prep_edges(ei, ew, branch):
    src = jnp.pad(ei[1], (0, EPAD - E)) + branch * N
    dst = jnp.pad(ei[0], (0, EPAD - E))
    eww = jnp.pad(ew, (0, EPAD - E))
    return (src.reshape(NTILE, NBLK, BCH, CHUNK),
            dst.reshape(NTILE, NBLK, BCH, CHUNK),
            eww.reshape(NTILE, NBLK, BCH, CHUNK))


def kernel(h, edge_index0, edge_index1, edge_weight0, edge_weight1, node_type,
           W0, a0, Wg0, bg0, Wb0, bb0, bias0, W1, a1, Wg1, bg1, Wb1, bb1,
           bias1, attW1, attb1, attW2):
    Ws = jnp.stack([W0, W1])
    seq = _seq_fts(h, Ws)

    s0, d0, w0 = _prep_edges(edge_index0, edge_weight0, 0)
    s1, d1, w1 = _prep_edges(edge_index1, edge_weight1, 1)
    src_s = jnp.stack([s0, s1])
    dst_s = jnp.stack([d0, d1])
    ew_s = jnp.stack([w0, w1])
    zeros = jnp.zeros((NPAD, D), jnp.float32)

    agg = _sc_agg(seq.reshape(2 * N, D), src_s, dst_s, ew_s, zeros)[:, :N]

    nt_f = node_type.astype(jnp.float32).reshape(N, 1)
    Wg_s = jnp.stack([Wg0, Wg1])
    Wb_s = jnp.stack([Wb0, Wb1])
    bv = jnp.stack([bg0, bb0, bias0, bg1, bb1, bias1,
                    jnp.zeros((D,), jnp.float32), jnp.zeros((D,), jnp.float32)])
    a_s = jnp.stack([a0, a1])
    return _film_att(h, seq, agg, nt_f, Wg_s, Wb_s, bv, a_s,
                     attW1, attb1.reshape(1, D), attW2)


# speedup vs baseline: 3.9742x; 1.2020x over previous
import functools

import jax
import jax.numpy as jnp
from jax import lax
from jax.experimental import pallas as pl
from jax.experimental.pallas import tpu as pltpu
from jax.experimental.pallas import tpu_sc as plsc

N = 10000
D = 128
E = 320000

NTILE = 16
CHUNK = 128
BCH = 8
NBLK = 20
NCH = NBLK * BCH
EPT = NCH * CHUNK
EPAD = EPT * NTILE
NPAD = 10240
ROWS_PT = NPAD // NTILE

BR = 2000
NB = N // BR


def _mm_body(h_ref, w_ref, o_ref):
    o_ref[0] = jnp.dot(h_ref[...], w_ref[0], preferred_element_type=jnp.float32)
    o_ref[1] = jnp.dot(h_ref[...], w_ref[1], preferred_element_type=jnp.float32)


def _seq_fts(h, Ws):
    return pl.pallas_call(
        _mm_body,
        grid=(NB,),
        in_specs=[
            pl.BlockSpec((BR, D), lambda i: (i, 0)),
            pl.BlockSpec((2, D, D), lambda i: (0, 0, 0)),
        ],
        out_specs=pl.BlockSpec((2, BR, D), lambda i: (0, i, 0)),
        out_shape=jax.ShapeDtypeStruct((2, N, D), jnp.float32),
    )(h, Ws)


def _sc_body(seq_hbm, src_hbm, dst_hbm, ew_hbm, z_hbm, out_hbm,
             src_v, dst_v, ew_v, rows_a, rows_b, acc,
             esem, gsa, gsb, ssa, ssb):
    c = lax.axis_index("c")
    s = lax.axis_index("s")

    pltpu.sync_copy(z_hbm.at[pl.ds(s * ROWS_PT, ROWS_PT)],
                    acc.at[pl.ds(s * ROWS_PT, ROWS_PT)])
    plsc.subcore_barrier()

    def g_issue(j, rows, sem):
        pltpu.async_copy(seq_hbm.at[src_v.at[j]], rows, sem)

    def g_wait(rows, sem):
        pltpu.make_async_copy(seq_hbm.at[src_v.at[0]], rows, sem).wait()

    def s_issue(j, rows, sem):
        pltpu.async_copy(rows, acc.at[dst_v.at[j]], sem, add=True)

    def s_wait(rows, sem):
        pltpu.make_async_copy(rows, acc.at[dst_v.at[0]], sem).wait()

    def scale(j, rows):
        def body(g, _):
            ew16 = ew_v[j, pl.ds(g * 16, 16)]
            for l in range(16):
                w = ew16[l]
                r = g * 16 + l
                for q in range(8):
                    sl = pl.ds(q * 16, 16)
                    rows[r, sl] = rows[r, sl] * w
            return 0

        lax.fori_loop(0, CHUNK // 16, body, 0)

    def block(b, _):
        pltpu.async_copy(src_hbm.at[c, s, b], src_v, esem)
        pltpu.async_copy(dst_hbm.at[c, s, b], dst_v, esem)
        pltpu.async_copy(ew_hbm.at[c, s, b], ew_v, esem)
        pltpu.make_async_copy(src_hbm.at[c, s, b], src_v, esem).wait()
        pltpu.make_async_copy(dst_hbm.at[c, s, b], dst_v, esem).wait()
        pltpu.make_async_copy(ew_hbm.at[c, s, b], ew_v, esem).wait()

        g_issue(0, rows_a, gsa)
        g_issue(1, rows_b, gsb)
        g_wait(rows_a, gsa)
        scale(0, rows_a)
        s_issue(0, rows_a, ssa)

        def pair(j2, _):
            ja = 2 * j2 + 1
            g_wait(rows_b, gsb)
            s_wait(rows_a, ssa)
            g_issue(ja + 1, rows_a, gsa)
            scale(ja, rows_b)
            s_issue(ja, rows_b, ssb)

            g_wait(rows_a, gsa)
            s_wait(rows_b, ssb)
            g_issue(ja + 2, rows_b, gsb)
            scale(ja + 1, rows_a)
            s_issue(ja + 1, rows_a, ssa)
            return 0

        lax.fori_loop(0, BCH // 2 - 1, pair, 0)

        g_wait(rows_b, gsb)
        s_wait(rows_a, ssa)
        scale(BCH - 1, rows_b)
        s_issue(BCH - 1, rows_b, ssb)
        s_wait(rows_b, ssb)
        return 0

    lax.fori_loop(0, NBLK, block, 0)
    plsc.subcore_barrier()
    pltpu.sync_copy(acc.at[pl.ds(s * ROWS_PT, ROWS_PT)],
                    out_hbm.at[c, pl.ds(s * ROWS_PT, ROWS_PT)])


def _sc_agg(seq2, src_s, dst_s, ew_s, zeros):
    mesh = plsc.VectorSubcoreMesh(core_axis_name="c", subcore_axis_name="s")
    k = functools.partial(
        pl.kernel,
        out_type=jax.ShapeDtypeStruct((2, NPAD, D), jnp.float32),
        mesh=mesh,
        scratch_types=[
            pltpu.VMEM((BCH, CHUNK), jnp.int32),
            pltpu.VMEM((BCH, CHUNK), jnp.int32),
            pltpu.VMEM((BCH, CHUNK), jnp.float32),
            pltpu.VMEM((CHUNK, D), jnp.float32),
            pltpu.VMEM((CHUNK, D), jnp.float32),
            pltpu.VMEM_SHARED((NPAD, D), jnp.float32),
            pltpu.SemaphoreType.DMA,
            pltpu.SemaphoreType.DMA,
            pltpu.SemaphoreType.DMA,
            pltpu.SemaphoreType.DMA,
            pltpu.SemaphoreType.DMA,
        ],
    )(_sc_body)
    return k(seq2, src_s, dst_s, ew_s, zeros)


def _film_body(h_ref, seq_ref, agg_ref, nt_ref, Wg_ref, Wb_ref, bv_ref,
               a_ref, aW1_ref, ab1_ref, aW2_ref, o_ref):
    nt = nt_ref[...]
    is0 = nt < 0.5
    zs = []
    for i in range(2):
        gam = jnp.where(is0, Wg_ref[i, 0][None, :], Wg_ref[i, 1][None, :])
        bet = jnp.where(is0, Wb_ref[i, 0][None, :], Wb_ref[i, 1][None, :])
        gam = gam + bv_ref[3 * i][None, :]
        bet = bet + bv_ref[3 * i + 1][None, :]
        z = gam * agg_ref[i] + bet + bv_ref[3 * i + 2][None, :] + seq_ref[i]
        z = jnp.where(z >= 0, z, a_ref[i] * z)
        zs.append(z)
    z0, z1 = zs
    aW1 = aW1_ref[...]
    ab1 = ab1_ref[...]
    aW2 = aW2_ref[...]
    w0 = jnp.dot(jnp.tanh(jnp.dot(z0, aW1, preferred_element_type=jnp.float32)
                          + ab1), aW2, preferred_element_type=jnp.float32)
    w1 = jnp.dot(jnp.tanh(jnp.dot(z1, aW1, preferred_element_type=jnp.float32)
                          + ab1), aW2, preferred_element_type=jnp.float32)
    m = jnp.maximum(w0, w1)
    e0 = jnp.exp(w0 - m)
    e1 = jnp.exp(w1 - m)
    inv = 1.0 / (e0 + e1)
    o_ref[...] = (e0 * inv) * z0 + (e1 * inv) * z1 + h_ref[...]


def _film_att(h, seq, agg, nt_f, Wg_s, Wb_s, bv, a_s, attW1, attb1, attW2):
    return pl.pallas_call(
        _film_body,
        grid=(NB,),
        in_specs=[
            pl.BlockSpec((BR, D), lambda i: (i, 0)),
            pl.BlockSpec((2, BR, D), lambda i: (0, i, 0)),
            pl.BlockSpec((2, BR, D), lambda i: (0, i, 0)),
            pl.BlockSpec((BR, 1), lambda i: (i, 0)),
            pl.BlockSpec((2, 2, D), lambda i: (0, 0, 0)),
            pl.BlockSpec((2, 2, D), lambda i: (0, 0, 0)),
            pl.BlockSpec((8, D), lambda i: (0, 0)),
            pl.BlockSpec(memory_space=pltpu.SMEM),
            pl.BlockSpec((D, D), lambda i: (0, 0)),
            pl.BlockSpec((1, D), lambda i: (0, 0)),
            pl.BlockSpec((D, 1), lambda i: (0, 0)),
        ],
        out_specs=pl.BlockSpec((BR, D), lambda i: (i, 0)),
        out_shape=jax.ShapeDtypeStruct((N, D), jnp.float32),
    )(h, seq, agg, nt_f, Wg_s, Wb_s, bv, a_s, attW1, attb1, attW2)


def _prep_edges(ei, ew, branch):
    src = jnp.pad(ei[1], (0, EPAD - E)) + branch * N
    dst = jnp.pad(ei[0], (0, EPAD - E))
    eww = jnp.pad(ew, (0, EPAD - E))
    return (src.reshape(NTILE, NBLK, BCH, CHUNK),
            dst.reshape(NTILE, NBLK, BCH, CHUNK),
            eww.reshape(NTILE, NBLK, BCH, CHUNK))


def kernel(h, edge_index0, edge_index1, edge_weight0, edge_weight1, node_type,
           W0, a0, Wg0, bg0, Wb0, bb0, bias0, W1, a1, Wg1, bg1, Wb1, bb1,
           bias1, attW1, attb1, attW2):
    Ws = jnp.stack([W0, W1])
    seq = _seq_fts(h, Ws)

    s0, d0, w0 = _prep_edges(edge_index0, edge_weight0, 0)
    s1, d1, w1 = _prep_edges(edge_index1, edge_weight1, 1)
    src_s = jnp.stack([s0, s1])
    dst_s = jnp.stack([d0, d1])
    ew_s = jnp.stack([w0, w1])
    zeros = jnp.zeros((NPAD, D), jnp.float32)

    agg = _sc_agg(seq.reshape(2 * N, D), src_s, dst_s, ew_s, zeros)[:, :N]

    nt_f = node_type.astype(jnp.float32).reshape(N, 1)
    Wg_s = jnp.stack([Wg0, Wg1])
    Wb_s = jnp.stack([Wb0, Wb1])
    bv = jnp.stack([bg0, bb0, bias0, bg1, bb1, bias1,
                    jnp.zeros((D,), jnp.float32), jnp.zeros((D,), jnp.float32)])
    a_s = jnp.stack([a0, a1])
    return _film_att(h, seq, agg, nt_f, Wg_s, Wb_s, bv, a_s,
                     attW1, attb1.reshape(1, D), attW2)
